# baseline (speedup 1.0000x reference)
import functools

import jax
import jax.numpy as jnp
from jax import lax
from jax.experimental import pallas as pl
from jax.experimental.pallas import tpu as pltpu
from jax.experimental.pallas import tpu_sc as plsc

NUM_CORES = 2
NUM_SUBCORES = 16
NUM_WORKERS = NUM_CORES * NUM_SUBCORES
CHUNK = 128
L = 16
NBUF = 3
GLAG = 2


def _emb_body(n_tasks, nh, nbt, x_hbm, tbl_hbm, out_hbm,
              idx_v, g_v, q_v, gb_v, tb_v, isem, gsem, ssem):
  wid = lax.axis_index("s") * NUM_CORES + lax.axis_index("c")
  base = wid * n_tasks

  def task_hb(t):
    tt = base + t
    return tt // nbt, tt % nbt

  def idx_dma(t, b):
    h, bt = task_hb(t)
    return pltpu.make_async_copy(
        x_hbm.at[h, pl.ds(bt * CHUNK, CHUNK)], idx_v.at[b], isem.at[b])

  def gather_dma(t, b):
    return pltpu.make_async_copy(tbl_hbm.at[g_v.at[b]], gb_v.at[b],
                                 gsem.at[b])

  def store_dma(t, b):
    h, bt = task_hb(t)
    return pltpu.make_async_copy(
        tb_v.at[b, :, pl.ds(0, CHUNK)],
        out_hbm.at[h, :, pl.ds(bt * CHUNK, CHUNK)], ssem.at[b])

  for t in range(GLAG):
    idx_dma(t, t % NBUF).start()

  iota16 = lax.iota(jnp.int32, L)
  zero16 = jnp.zeros((L,), jnp.int32)
  jlanes = [iota16 + j0 for j0 in range(0, 32, L)]

  @pl.loop(0, n_tasks + GLAG, step=NBUF)
  def _(g0):
    for b in range(NBUF):
      t = g0 + b

      @pl.when(t < n_tasks)
      def _():
        idx_dma(t, b).wait()
        for l in range(CHUNK // L):
          v = idx_v.at[b][pl.ds(l * L, L)]
          g_v.at[b][pl.ds(l * L, L)] = lax.shift_right_logical(v, 2)
          q_v.at[b][pl.ds(l * L, L)] = lax.shift_left(
              lax.bitwise_and(v, 3), 5)
        gather_dma(t, b).start()

        @pl.when(t + GLAG < n_tasks)
        def _():
          idx_dma(t + GLAG, (b + GLAG) % NBUF).start()

      u = t - GLAG
      bu = (b - GLAG) % NBUF

      @pl.when(jnp.logical_and(u >= 0, u < n_tasks))
      def _():
        @pl.when(u >= NBUF)
        def _():
          store_dma(u - NBUF, bu).wait()

        gather_dma(u, bu).wait()

        @pl.loop(0, CHUNK, unroll=8)
        def _(r):
          rsp = zero16 + r
          qb = plsc.load_gather(q_v.at[bu], [rsp])
          for half in range(2):
            val = plsc.load_gather(gb_v.at[bu], [rsp, qb + jlanes[half]])
            plsc.store_scatter(tb_v.at[bu], [jlanes[half], rsp], val)

        store_dma(u, bu).start()

  for k in range(NBUF):
    u = n_tasks - NBUF + k
    store_dma(u, u % NBUF).wait()


def kernel(x, table):
  bsz, nh = x.shape
  nv, d = table.shape
  nbt = bsz // CHUNK
  n_tasks = nh * nbt // NUM_WORKERS
  lines = nv * d // 128

  x_t = x.T.astype(jnp.int32)
  tbl = table.reshape(lines, 128)

  mesh = plsc.VectorSubcoreMesh(
      core_axis_name="c", subcore_axis_name="s", num_cores=NUM_CORES,
      num_subcores=NUM_SUBCORES)

  emb = pl.kernel(
      functools.partial(_emb_body, n_tasks, nh, nbt),
      out_type=jax.ShapeDtypeStruct((nh, d, bsz), jnp.float32),
      mesh=mesh,
      scratch_types=[
          pltpu.VMEM((NBUF, CHUNK), jnp.int32),
          pltpu.VMEM((NBUF, CHUNK), jnp.int32),
          pltpu.VMEM((NBUF, CHUNK), jnp.int32),
          pltpu.VMEM((NBUF, CHUNK, 128), jnp.float32),
          pltpu.VMEM((NBUF, d, CHUNK + 1), jnp.float32),
          pltpu.SemaphoreType.DMA((NBUF,)),
          pltpu.SemaphoreType.DMA((NBUF,)),
          pltpu.SemaphoreType.DMA((NBUF,)),
      ],
      compiler_params=pltpu.CompilerParams(
          use_tc_tiling_on_sc=True,
          needs_layout_passes=False,
          disable_bounds_checks=True,
          disable_semaphore_checks=True,
      ),
  )
  out = emb(x_t, tbl)
  return out.transpose(2, 0, 1)


# speedup vs baseline: 1.0355x; 1.0355x over previous
import functools

import jax
import jax.numpy as jnp
from jax import lax
from jax.experimental import pallas as pl
from jax.experimental.pallas import tpu as pltpu
from jax.experimental.pallas import tpu_sc as plsc

NUM_CORES = 2
NUM_SUBCORES = 16
NUM_WORKERS = NUM_CORES * NUM_SUBCORES
CHUNK = 128
L = 16
NBUF = 3
GLAG = 2


def _emb_body(n_tasks, nh, nbt, x_hbm, tbl_hbm, out_hbm,
              idx_v, g_v, q_v, gb_v, tb_v, isem, gsem, ssem):
  wid = lax.axis_index("s") * NUM_CORES + lax.axis_index("c")
  base = wid * n_tasks

  def task_hb(t):
    tt = base + t
    return tt // nbt, tt % nbt

  def idx_dma(t, b):
    h, bt = task_hb(t)
    return pltpu.make_async_copy(
        x_hbm.at[h, pl.ds(bt * CHUNK, CHUNK)], idx_v.at[b], isem.at[b])

  def gather_dma(t, b):
    return pltpu.make_async_copy(tbl_hbm.at[g_v.at[b]],
                                 gb_v.at[b, :, pl.ds(0, 128)], gsem.at[b])

  def store_dma(t, b):
    h, bt = task_hb(t)
    return pltpu.make_async_copy(
        tb_v.at[b], out_hbm.at[h, :, pl.ds(bt * CHUNK, CHUNK)], ssem.at[b])

  for t in range(GLAG):
    idx_dma(t, t % NBUF).start()

  iota16 = lax.iota(jnp.int32, L)
  rows = [iota16 + l * L for l in range(CHUNK // L)]

  @pl.loop(0, n_tasks + GLAG, step=NBUF)
  def _(g0):
    for b in range(NBUF):
      t = g0 + b

      @pl.when(t < n_tasks)
      def _():
        idx_dma(t, b).wait()
        for l in range(CHUNK // L):
          v = idx_v.at[b][pl.ds(l * L, L)]
          g_v.at[b][pl.ds(l * L, L)] = lax.shift_right_logical(v, 2)
          q_v.at[b][pl.ds(l * L, L)] = lax.shift_left(
              lax.bitwise_and(v, 3), 5)
        gather_dma(t, b).start()

        @pl.when(t + GLAG < n_tasks)
        def _():
          idx_dma(t + GLAG, (b + GLAG) % NBUF).start()

      u = t - GLAG
      bu = (b - GLAG) % NBUF

      @pl.when(jnp.logical_and(u >= 0, u < n_tasks))
      def _():
        @pl.when(u >= NBUF)
        def _():
          store_dma(u - NBUF, bu).wait()

        gather_dma(u, bu).wait()

        for l in range(CHUNK // L):
          q = q_v.at[bu][pl.ds(l * L, L)]
          for w in range(32):
            val = plsc.load_gather(gb_v.at[bu], [rows[l], q + w])
            tb_v.at[bu][w, pl.ds(l * L, L)] = val

        store_dma(u, bu).start()

  for k in range(NBUF):
    u = n_tasks - NBUF + k
    store_dma(u, u % NBUF).wait()


def kernel(x, table):
  bsz, nh = x.shape
  nv, d = table.shape
  nbt = bsz // CHUNK
  n_tasks = nh * nbt // NUM_WORKERS
  lines = nv * d // 128

  x_t = x.T.astype(jnp.int32)
  tbl = table.reshape(lines, 128)

  mesh = plsc.VectorSubcoreMesh(
      core_axis_name="c", subcore_axis_name="s", num_cores=NUM_CORES,
      num_subcores=NUM_SUBCORES)

  emb = pl.kernel(
      functools.partial(_emb_body, n_tasks, nh, nbt),
      out_type=jax.ShapeDtypeStruct((nh, d, bsz), jnp.float32),
      mesh=mesh,
      scratch_types=[
          pltpu.VMEM((NBUF, CHUNK), jnp.int32),
          pltpu.VMEM((NBUF, CHUNK), jnp.int32),
          pltpu.VMEM((NBUF, CHUNK), jnp.int32),
          pltpu.VMEM((NBUF, CHUNK, 129), jnp.float32),
          pltpu.VMEM((NBUF, d, CHUNK), jnp.float32),
          pltpu.SemaphoreType.DMA((NBUF,)),
          pltpu.SemaphoreType.DMA((NBUF,)),
          pltpu.SemaphoreType.DMA((NBUF,)),
      ],
      compiler_params=pltpu.CompilerParams(
          use_tc_tiling_on_sc=True,
          needs_layout_passes=False,
          disable_bounds_checks=True,
          disable_semaphore_checks=True,
      ),
  )
  out = emb(x_t, tbl)
  return out.transpose(2, 0, 1)


# speedup vs baseline: 1.2685x; 1.2250x over previous
import functools

import jax
import jax.numpy as jnp
from jax import lax
from jax.experimental import pallas as pl
from jax.experimental.pallas import tpu as pltpu
from jax.experimental.pallas import tpu_sc as plsc

NUM_CORES = 2
NUM_SUBCORES = 16
NUM_WORKERS = NUM_CORES * NUM_SUBCORES
CHUNK = 128
NBUF = 8
GDEPTH = NBUF - 2


def _emb_kernel_body(n_chunks, d, x_hbm, table_hbm, out_hbm, idx_v, rows_v,
                     gsem, ssem):
  wid = lax.axis_index("s") * NUM_CORES + lax.axis_index("c")
  base = wid * n_chunks
  pltpu.sync_copy(x_hbm.at[wid], idx_v)

  def out_at(j):
    return out_hbm.at[pl.ds((base + j) * CHUNK, CHUNK), :]

  for b in range(GDEPTH):
    pltpu.async_copy(table_hbm.at[idx_v.at[b]], rows_v.at[b], gsem.at[b])

  @pl.loop(0, n_chunks, step=NBUF)
  def _(g):
    for b in range(NBUF):
      j = g + b
      pltpu.make_async_copy(table_hbm.at[idx_v.at[j]], rows_v.at[b],
                            gsem.at[b]).wait()
      pltpu.async_copy(rows_v.at[b], out_at(j), ssem.at[b])

      nxt = j + GDEPTH
      b2 = (b + GDEPTH) % NBUF

      @pl.when(jnp.logical_and(nxt < n_chunks, nxt >= NBUF))
      def _():
        pltpu.make_async_copy(rows_v.at[b2], out_at(nxt - NBUF),
                              ssem.at[b2]).wait()

      @pl.when(nxt < n_chunks)
      def _():
        pltpu.async_copy(table_hbm.at[idx_v.at[nxt]], rows_v.at[b2],
                         gsem.at[b2])

  for b in range(NBUF):
    pltpu.make_async_copy(rows_v.at[b], out_at(n_chunks - NBUF + b),
                          ssem.at[b]).wait()


def kernel(x, table):
  b, h = x.shape
  _, d = table.shape
  n = b * h
  assert n % (NUM_WORKERS * CHUNK) == 0
  n_chunks = n // (NUM_WORKERS * CHUNK)

  x_flat = x.reshape(NUM_WORKERS, n_chunks, CHUNK).astype(jnp.int32)

  mesh = plsc.VectorSubcoreMesh(
      core_axis_name="c", subcore_axis_name="s", num_cores=NUM_CORES,
      num_subcores=NUM_SUBCORES)

  emb = pl.kernel(
      functools.partial(_emb_kernel_body, n_chunks, d),
      out_type=jax.ShapeDtypeStruct((n, d), jnp.float32),
      mesh=mesh,
      scratch_types=[
          pltpu.VMEM((n_chunks, CHUNK), jnp.int32),
          pltpu.VMEM((NBUF, CHUNK, d), jnp.float32),
          pltpu.SemaphoreType.DMA((NBUF,)),
          pltpu.SemaphoreType.DMA((NBUF,)),
      ],
      compiler_params=pltpu.CompilerParams(use_tc_tiling_on_sc=False),
  )
  out = emb(x_flat, table)
  return out.reshape(b, h, d)


# speedup vs baseline: 1.2690x; 1.0004x over previous
import functools

import jax
import jax.numpy as jnp
from jax import lax
from jax.experimental import pallas as pl
from jax.experimental.pallas import tpu as pltpu
from jax.experimental.pallas import tpu_sc as plsc

NUM_CORES = 2
NUM_SUBCORES = 16
NUM_WORKERS = NUM_CORES * NUM_SUBCORES
CHUNK = 128
NBUF = 10
GDEPTH = NBUF - 2


def _emb_kernel_body(n_chunks, d, x_hbm, table_hbm, out_hbm, idx_v, rows_v,
                     gsem, ssem):
  wid = lax.axis_index("s") * NUM_CORES + lax.axis_index("c")
  base = wid * n_chunks
  pltpu.sync_copy(x_hbm.at[wid], idx_v)

  def out_at(j):
    return out_hbm.at[pl.ds((base + j) * CHUNK, CHUNK), :]

  for b in range(GDEPTH):
    pltpu.async_copy(table_hbm.at[idx_v.at[b]], rows_v.at[b], gsem.at[b])

  @pl.loop(0, n_chunks, step=NBUF)
  def _(g):
    for b in range(NBUF):
      j = g + b
      pltpu.make_async_copy(table_hbm.at[idx_v.at[j]], rows_v.at[b],
                            gsem.at[b]).wait()
      pltpu.async_copy(rows_v.at[b], out_at(j), ssem.at[b])

      nxt = j + GDEPTH
      b2 = (b + GDEPTH) % NBUF

      @pl.when(jnp.logical_and(nxt < n_chunks, nxt >= NBUF))
      def _():
        pltpu.make_async_copy(rows_v.at[b2], out_at(nxt - NBUF),
                              ssem.at[b2]).wait()

      @pl.when(nxt < n_chunks)
      def _():
        pltpu.async_copy(table_hbm.at[idx_v.at[nxt]], rows_v.at[b2],
                         gsem.at[b2])

  for b in range(NBUF):
    pltpu.make_async_copy(rows_v.at[b], out_at(n_chunks - NBUF + b),
                          ssem.at[b]).wait()


def kernel(x, table):
  b, h = x.shape
  _, d = table.shape
  n = b * h
  assert n % (NUM_WORKERS * CHUNK) == 0
  n_chunks = n // (NUM_WORKERS * CHUNK)

  x_flat = x.reshape(NUM_WORKERS, n_chunks, CHUNK).astype(jnp.int32)

  mesh = plsc.VectorSubcoreMesh(
      core_axis_name="c", subcore_axis_name="s", num_cores=NUM_CORES,
      num_subcores=NUM_SUBCORES)

  emb = pl.kernel(
      functools.partial(_emb_kernel_body, n_chunks, d),
      out_type=jax.ShapeDtypeStruct((n, d), jnp.float32),
      mesh=mesh,
      scratch_types=[
          pltpu.VMEM((n_chunks, CHUNK), jnp.int32),
          pltpu.VMEM((NBUF, CHUNK, d), jnp.float32),
          pltpu.SemaphoreType.DMA((NBUF,)),
          pltpu.SemaphoreType.DMA((NBUF,)),
      ],
      compiler_params=pltpu.CompilerParams(use_tc_tiling_on_sc=False),
  )
  out = emb(x_flat, table)
  return out.reshape(b, h, d)
